# baseline (speedup 1.0000x reference)
import jax
import jax.numpy as jnp
from jax import lax
from jax.experimental import pallas as pl
from jax.experimental.pallas import tpu as pltpu

N_ROWS = 100000
PACK_ROWS = 3125
PACK_COLS = 384


def _kern(x_ref, wa1_ref, ba1_ref, wa5_ref, ba5_ref, out_ref):
    sums = jnp.sum(x_ref[...], axis=0, keepdims=True)
    lane = lax.broadcasted_iota(jnp.int32, (PACK_COLS, 12), 0)
    feat = lax.broadcasted_iota(jnp.int32, (PACK_COLS, 12), 1)
    onehot = (lane % 12 == feat).astype(jnp.float32)
    x_prime = jnp.dot(
        sums, onehot, preferred_element_type=jnp.float32
    ) * (2.0 / N_ROWS)
    a1 = jnp.sum(wa1_ref[...] * x_prime, axis=1, keepdims=True).T
    a1 = jnp.maximum(a1 + ba1_ref[...], 0.0)
    out_ref[...] = (
        jnp.sum(a1 * wa5_ref[...], axis=1, keepdims=True) + ba5_ref[...]
    )


def kernel(x, edge_index, W1_rel, b1_rel, W1_root, W2_rel, b2_rel, W2_root,
           Wa1, ba1, Wa5, ba5):
    del edge_index, W1_rel, b1_rel, W1_root, W2_rel, b2_rel, W2_root
    xp = x.reshape(PACK_ROWS, PACK_COLS)
    return pl.pallas_call(
        _kern,
        out_shape=jax.ShapeDtypeStruct((1, 1), jnp.float32),
    )(xp, Wa1, ba1.reshape(1, 11), Wa5, ba5.reshape(1, 1))
